# Initial kernel scaffold; baseline (speedup 1.0000x reference)
import jax
import jax.numpy as jnp
from jax import lax
from jax.experimental import pallas as pl
from jax.experimental.pallas import tpu as pltpu
from jax.experimental.pallas import tpu_sc as plsc

N = 10000
E = 320000
F = 128
G = 100
NCLS = 3
NCORE = 2
NSUB = 16
NW = NCORE * NSUB
EPW = E // NW
B = 125
NB = EPW // B
ER = E // B
RPS = N // NSUB


def _segsum_sc(h, zeros, src_rows, dst_rows):
  mesh = plsc.VectorSubcoreMesh(core_axis_name="c", subcore_axis_name="s",
                                num_cores=NCORE, num_subcores=NSUB)

  def body(h_hbm, z_hbm, src_hbm, dst_hbm, out_hbm, src_v, dst_v, rows_v,
           acc, sem):
    c = lax.axis_index("c")
    s = lax.axis_index("s")
    r0 = s * RPS

    @pl.when(c == 0)
    def _():
      pltpu.sync_copy(h_hbm.at[pl.ds(r0, RPS)], acc.at[pl.ds(r0, RPS)])

    @pl.when(c != 0)
    def _():
      pltpu.sync_copy(z_hbm.at[pl.ds(r0, RPS)], acc.at[pl.ds(r0, RPS)])

    w = c * NSUB + s
    pltpu.sync_copy(src_hbm.at[pl.ds(w * NB, NB)], src_v)
    pltpu.sync_copy(dst_hbm.at[pl.ds(w * NB, NB)], dst_v)
    plsc.subcore_barrier()

    def step(j, carry):
      pltpu.async_copy(h_hbm.at[src_v.at[j]], rows_v, sem).wait()
      pltpu.sync_copy(rows_v, acc.at[dst_v.at[j]], add=True)
      return carry

    lax.fori_loop(0, NB, step, 0)
    plsc.subcore_barrier()
    pltpu.sync_copy(acc.at[pl.ds(r0, RPS)], out_hbm.at[c, pl.ds(r0, RPS)])

  f = pl.kernel(
      body,
      out_type=jax.ShapeDtypeStruct((NCORE, N, F), jnp.float32),
      mesh=mesh,
      scratch_types=[
          pltpu.VMEM((NB, B), jnp.int32),
          pltpu.VMEM((NB, B), jnp.int32),
          pltpu.VMEM((B, F), jnp.float32),
          pltpu.VMEM_SHARED((N, F), jnp.float32),
          pltpu.SemaphoreType.DMA,
      ],
  )
  return f(h, zeros, src_rows, dst_rows)


def _dot(a, b):
  return lax.dot_general(a, b, (((1,), (0,)), ((), ())),
                         precision=lax.Precision.HIGHEST,
                         preferred_element_type=jnp.float32)


def _mlp_tc(p, Wa, ba, Wb, bb, scale, shift):
  BR = 1000

  def body(p_ref, wa_ref, ba_ref, wb_ref, bb_ref, g_ref, b_ref, o_ref):
    t = p_ref[0] + p_ref[1]
    t = jnp.maximum(_dot(t, wa_ref[...]) + ba_ref[...], 0.0)
    t = _dot(t, wb_ref[...]) + bb_ref[...]
    t = jnp.maximum(t, 0.0)
    o_ref[...] = t * g_ref[...] + b_ref[...]

  return pl.pallas_call(
      body,
      grid=(N // BR,),
      in_specs=[
          pl.BlockSpec((NCORE, BR, F), lambda i: (0, i, 0)),
          pl.BlockSpec((F, F), lambda i: (0, 0)),
          pl.BlockSpec((1, F), lambda i: (0, 0)),
          pl.BlockSpec((F, F), lambda i: (0, 0)),
          pl.BlockSpec((1, F), lambda i: (0, 0)),
          pl.BlockSpec((1, F), lambda i: (0, 0)),
          pl.BlockSpec((1, F), lambda i: (0, 0)),
      ],
      out_specs=pl.BlockSpec((BR, F), lambda i: (i, 0)),
      out_shape=jax.ShapeDtypeStruct((N, F), jnp.float32),
  )(p, Wa, ba, Wb, bb, scale, shift)


def _final_tc(h3, Wf, bf):
  def body(h_ref, wf_ref, bf_ref, o_ref):
    r = lax.broadcasted_iota(jnp.int32, (G, N), 0)
    cgrp = lax.broadcasted_iota(jnp.int32, (G, N), 1) // (N // G)
    pool = (r == cgrp).astype(jnp.float32)
    tp = _dot(pool, h_ref[...])
    o_ref[...] = _dot(tp, wf_ref[...]) + bf_ref[...]

  return pl.pallas_call(
      body,
      out_shape=jax.ShapeDtypeStruct((G, NCLS), jnp.float32),
  )(h3, Wf, bf)


def kernel(x, edge_index, num_graphs, W1a, b1a, W1b, b1b, g1, be1,
           W2a, b2a, W2b, b2b, g2, be2, W3a, b3a, W3b, b3b, g3, be3, Wf, bf):
  src_rows = edge_index[0].reshape(ER, B)
  dst_rows = edge_index[1].reshape(ER, B)
  zeros = jnp.zeros((N, F), jnp.float32)
  inv = jnp.float32(1.0) / jnp.sqrt(jnp.float32(1.0 + 1e-5))

  h = x
  for Wa, ba, Wb, bb, g, be in ((W1a, b1a, W1b, b1b, g1, be1),
                                (W2a, b2a, W2b, b2b, g2, be2),
                                (W3a, b3a, W3b, b3b, g3, be3)):
    p = _segsum_sc(h, zeros, src_rows, dst_rows)
    h = _mlp_tc(p, Wa, ba.reshape(1, F), Wb, bb.reshape(1, F),
                (g * inv).reshape(1, F), be.reshape(1, F))
  return _final_tc(h, Wf, bf.reshape(1, NCLS))


# speedup vs baseline: 6.7251x; 6.7251x over previous
import jax
import jax.numpy as jnp
from jax import lax
from jax.experimental import pallas as pl
from jax.experimental.pallas import tpu as pltpu
from jax.experimental.pallas import tpu_sc as plsc

N = 10000
E = 320000
F = 128
G = 100
NCLS = 3
NCORE = 2
NSUB = 16
NW = NCORE * NSUB
EPW = E // NW
B = 125
NB = EPW // B
ER = E // B
RPS_BIG = 640
RPS_LAST = N - (NSUB - 1) * RPS_BIG


def _segsum_sc(h, zeros, src_rows, dst_rows):
  mesh = plsc.VectorSubcoreMesh(core_axis_name="c", subcore_axis_name="s",
                                num_cores=NCORE, num_subcores=NSUB)

  def body(h_hbm, z_hbm, src_hbm, dst_hbm, out_hbm, src_v, dst_v, rows_v,
           acc, sem):
    c = lax.axis_index("c")
    s = lax.axis_index("s")
    r0 = s * RPS_BIG

    for nrows, pred in ((RPS_BIG, s < NSUB - 1), (RPS_LAST, s == NSUB - 1)):
      @pl.when(jnp.logical_and(c == 0, pred))
      def _(nrows=nrows):
        pltpu.sync_copy(h_hbm.at[pl.ds(r0, nrows)], acc.at[pl.ds(r0, nrows)])

      @pl.when(jnp.logical_and(c != 0, pred))
      def _(nrows=nrows):
        pltpu.sync_copy(z_hbm.at[pl.ds(r0, nrows)], acc.at[pl.ds(r0, nrows)])

    w = c * NSUB + s
    pltpu.sync_copy(src_hbm.at[pl.ds(w * NB, NB)], src_v)
    pltpu.sync_copy(dst_hbm.at[pl.ds(w * NB, NB)], dst_v)
    plsc.subcore_barrier()

    def step(j, carry):
      pltpu.async_copy(h_hbm.at[src_v.at[j]], rows_v, sem).wait()
      pltpu.sync_copy(rows_v, acc.at[dst_v.at[j]], add=True)
      return carry

    lax.fori_loop(0, NB, step, 0)
    plsc.subcore_barrier()
    for nrows, pred in ((RPS_BIG, s < NSUB - 1), (RPS_LAST, s == NSUB - 1)):
      @pl.when(pred)
      def _(nrows=nrows):
        pltpu.sync_copy(acc.at[pl.ds(r0, nrows)],
                        out_hbm.at[c, pl.ds(r0, nrows)])

  f = pl.kernel(
      body,
      out_type=jax.ShapeDtypeStruct((NCORE, N, F), jnp.float32),
      mesh=mesh,
      scratch_types=[
          pltpu.VMEM((NB, B), jnp.int32),
          pltpu.VMEM((NB, B), jnp.int32),
          pltpu.VMEM((B, F), jnp.float32),
          pltpu.VMEM_SHARED((N, F), jnp.float32),
          pltpu.SemaphoreType.DMA,
      ],
  )
  return f(h, zeros, src_rows, dst_rows)


def _dot(a, b):
  return lax.dot_general(a, b, (((1,), (0,)), ((), ())),
                         precision=lax.Precision.HIGHEST,
                         preferred_element_type=jnp.float32)


def _mlp_tc(p, Wa, ba, Wb, bb, scale, shift):
  BR = 1000

  def body(p_ref, wa_ref, ba_ref, wb_ref, bb_ref, g_ref, b_ref, o_ref):
    t = p_ref[0] + p_ref[1]
    t = jnp.maximum(_dot(t, wa_ref[...]) + ba_ref[...], 0.0)
    t = _dot(t, wb_ref[...]) + bb_ref[...]
    t = jnp.maximum(t, 0.0)
    o_ref[...] = t * g_ref[...] + b_ref[...]

  return pl.pallas_call(
      body,
      grid=(N // BR,),
      in_specs=[
          pl.BlockSpec((NCORE, BR, F), lambda i: (0, i, 0)),
          pl.BlockSpec((F, F), lambda i: (0, 0)),
          pl.BlockSpec((1, F), lambda i: (0, 0)),
          pl.BlockSpec((F, F), lambda i: (0, 0)),
          pl.BlockSpec((1, F), lambda i: (0, 0)),
          pl.BlockSpec((1, F), lambda i: (0, 0)),
          pl.BlockSpec((1, F), lambda i: (0, 0)),
      ],
      out_specs=pl.BlockSpec((BR, F), lambda i: (i, 0)),
      out_shape=jax.ShapeDtypeStruct((N, F), jnp.float32),
  )(p, Wa, ba, Wb, bb, scale, shift)


def _final_tc(h3, Wf, bf):
  def body(h_ref, wf_ref, bf_ref, o_ref):
    r = lax.broadcasted_iota(jnp.int32, (G, N), 0)
    cgrp = lax.broadcasted_iota(jnp.int32, (G, N), 1) // (N // G)
    pool = (r == cgrp).astype(jnp.float32)
    tp = _dot(pool, h_ref[...])
    o_ref[...] = _dot(tp, wf_ref[...]) + bf_ref[...]

  return pl.pallas_call(
      body,
      out_shape=jax.ShapeDtypeStruct((G, NCLS), jnp.float32),
  )(h3, Wf, bf)


def kernel(x, edge_index, num_graphs, W1a, b1a, W1b, b1b, g1, be1,
           W2a, b2a, W2b, b2b, g2, be2, W3a, b3a, W3b, b3b, g3, be3, Wf, bf):
  src_rows = edge_index[0].reshape(ER, B)
  dst_rows = edge_index[1].reshape(ER, B)
  zeros = jnp.zeros((N, F), jnp.float32)
  inv = jnp.float32(1.0) / jnp.sqrt(jnp.float32(1.0 + 1e-5))

  h = x
  for Wa, ba, Wb, bb, g, be in ((W1a, b1a, W1b, b1b, g1, be1),
                                (W2a, b2a, W2b, b2b, g2, be2),
                                (W3a, b3a, W3b, b3b, g3, be3)):
    p = _segsum_sc(h, zeros, src_rows, dst_rows)
    h = _mlp_tc(p, Wa, ba.reshape(1, F), Wb, bb.reshape(1, F),
                (g * inv).reshape(1, F), be.reshape(1, F))
  return _final_tc(h, Wf, bf.reshape(1, NCLS))
